# baseline (speedup 1.0000x reference)
import functools

import jax
import jax.numpy as jnp
from jax import lax
from jax.experimental import pallas as pl
from jax.experimental.pallas import tpu as pltpu
from jax.experimental.pallas import tpu_sc as plsc

B = 2048
D = 2048
NP = 8
PE = 256
NC = 16
EPS = 1e-5

BB = 512

_DNT = (((1,), (1,)), ((), ()))

SC_CORES = 2
SC_SUBCORES = 16
NW = SC_CORES * SC_SUBCORES
BPW = B // NW


def _ln(v):
    n = v.shape[1]
    ones = jnp.ones((1, n), jnp.float32)
    ones_c = jnp.ones((n, 1), jnp.float32)
    m = jax.lax.dot_general(v, ones, _DNT) * (1.0 / n)
    s2 = jax.lax.dot_general(v * v, ones, _DNT) * (1.0 / n)
    inv = jax.lax.rsqrt(s2 - m * m + EPS)
    a = jax.lax.dot_general(inv, ones_c, _DNT)
    b = jax.lax.dot_general(-m * inv, ones_c, _DNT)
    return v * a + b


def _tc1_kernel(x_ref, wp_ref, wc_ref, wpp_ref, bp_ref, bpp_ref,
                pl_out, pl0t_out, clall_out, pp_out):
    xb = x_ref[...]
    pl0 = jax.lax.dot_general(xb, wp_ref[...], _DNT) + bp_ref[...]
    pl0t_out[...] = pl0.T
    clall_out[...] = jax.lax.dot_general(xb, wc_ref[...], _DNT)
    pln = _ln(pl0)
    pl_out[...] = pln
    pp_out[...] = jax.lax.dot_general(pln, wpp_ref[...], _DNT) + bpp_ref[...]


def _sc_route_kernel(pl0t_hbm, pc_hbm, buf, pcv):
    wid = lax.axis_index("s") * SC_CORES + lax.axis_index("c")
    base = wid * BPW
    for p in range(NP):
        pltpu.sync_copy(pl0t_hbm.at[pl.ds(p * B + base, BPW)],
                        buf.at[pl.ds(p * BPW, BPW)])
    for g in range(BPW // 16):
        off = g * 16
        best = buf[pl.ds(off, 16)]
        bidx = jnp.zeros((16,), jnp.int32)
        for p in range(1, NP):
            v = buf[pl.ds(p * BPW + off, 16)]
            gt = v > best
            best = jnp.where(gt, v, best)
            bidx = jnp.where(gt, jnp.full((16,), p, jnp.int32), bidx)
        pcv[pl.ds(off, 16)] = bidx
    pltpu.sync_copy(pcv, pc_hbm.at[pl.ds(base, BPW)])


def _tc2_kernel(clall_ref, pc_ref, wcp_ref, bc_ref, bcp_ref, cl_out, cp_out):
    pc = pc_ref[...]
    lane8 = jax.lax.broadcasted_iota(jnp.int32, (1, NP), 1)
    onehot8 = (pc == lane8).astype(jnp.float32)
    lane128 = jax.lax.broadcasted_iota(jnp.int32, (1, NP * NC), 1)
    mask128 = pc == (lane128 // NC)

    frow = jax.lax.broadcasted_iota(jnp.int32, (NC, NP * NC), 0)
    fcol = jax.lax.broadcasted_iota(jnp.int32, (NC, NP * NC), 1)
    fold = (fcol % NC == frow).astype(jnp.float32)

    clm = jnp.where(mask128, clall_ref[...], 0.0)
    cl16 = jax.lax.dot_general(clm, fold, _DNT)
    cl16 = cl16 + jax.lax.dot(onehot8, bc_ref[...])
    cln = _ln(cl16)
    cl_out[...] = cln
    rep = jax.lax.dot_general(cln, fold.T, _DNT)
    scat = jnp.where(mask128, rep, 0.0)
    cp = jax.lax.dot(scat, wcp_ref[...])
    cp_out[...] = cp + jax.lax.dot(onehot8, bcp_ref[...])


@jax.jit
def kernel(x, Wp, bp, Wpp, bpp, Wc, bc, Wcp, bcp):
    wc_flat = Wc.reshape(NP * NC, D)
    wcp_flat = jnp.transpose(Wcp, (0, 2, 1)).reshape(NP * NC, PE)

    pln, pl0t, cl_all, pp = pl.pallas_call(
        _tc1_kernel,
        grid=(B // BB,),
        in_specs=[
            pl.BlockSpec((BB, D), lambda i: (i, 0)),
            pl.BlockSpec((NP, D), lambda i: (0, 0)),
            pl.BlockSpec((NP * NC, D), lambda i: (0, 0)),
            pl.BlockSpec((PE, NP), lambda i: (0, 0)),
            pl.BlockSpec((1, NP), lambda i: (0, 0)),
            pl.BlockSpec((1, PE), lambda i: (0, 0)),
        ],
        out_specs=[
            pl.BlockSpec((BB, NP), lambda i: (i, 0)),
            pl.BlockSpec((NP, BB), lambda i: (0, i)),
            pl.BlockSpec((BB, NP * NC), lambda i: (i, 0)),
            pl.BlockSpec((BB, PE), lambda i: (i, 0)),
        ],
        out_shape=[
            jax.ShapeDtypeStruct((B, NP), jnp.float32),
            jax.ShapeDtypeStruct((NP, B), jnp.float32),
            jax.ShapeDtypeStruct((B, NP * NC), jnp.float32),
            jax.ShapeDtypeStruct((B, PE), jnp.float32),
        ],
    )(x, Wp, wc_flat, Wpp, bp[None, :], bpp[None, :])

    mesh = plsc.VectorSubcoreMesh(core_axis_name="c", subcore_axis_name="s")
    pc = pl.kernel(
        _sc_route_kernel,
        mesh=mesh,
        out_type=jax.ShapeDtypeStruct((B,), jnp.int32),
        scratch_types=[
            pltpu.VMEM((NP * BPW,), jnp.float32),
            pltpu.VMEM((BPW,), jnp.int32),
        ],
    )(pl0t.reshape(NP * B))

    cln, cp = pl.pallas_call(
        _tc2_kernel,
        grid=(B // BB,),
        in_specs=[
            pl.BlockSpec((BB, NP * NC), lambda i: (i, 0)),
            pl.BlockSpec((BB, 1), lambda i: (i, 0)),
            pl.BlockSpec((NP * NC, PE), lambda i: (0, 0)),
            pl.BlockSpec((NP, NC), lambda i: (0, 0)),
            pl.BlockSpec((NP, PE), lambda i: (0, 0)),
        ],
        out_specs=[
            pl.BlockSpec((BB, NC), lambda i: (i, 0)),
            pl.BlockSpec((BB, PE), lambda i: (i, 0)),
        ],
        out_shape=[
            jax.ShapeDtypeStruct((B, NC), jnp.float32),
            jax.ShapeDtypeStruct((B, PE), jnp.float32),
        ],
    )(cl_all, pc.reshape(B, 1), wcp_flat, bc, bcp)

    return (pln, cln, pp, cp)


# speedup vs baseline: 2.3272x; 2.3272x over previous
import jax
import jax.numpy as jnp
from jax.experimental import pallas as pl
from jax.experimental.pallas import tpu as pltpu

B = 2048
D = 2048
NP = 8
PE = 256
NC = 16
EPS = 1e-5

BB = 512

_DNT = (((1,), (1,)), ((), ()))


def _ln(v):
    n = v.shape[1]
    ones = jnp.ones((1, n), jnp.float32)
    ones_c = jnp.ones((n, 1), jnp.float32)
    m = jax.lax.dot_general(v, ones, _DNT) * (1.0 / n)
    s2 = jax.lax.dot_general(v * v, ones, _DNT) * (1.0 / n)
    inv = jax.lax.rsqrt(s2 - m * m + EPS)
    a = jax.lax.dot_general(inv, ones_c, _DNT)
    b = jax.lax.dot_general(-m * inv, ones_c, _DNT)
    return v * a + b


def _hc_kernel(x_ref, wp_ref, wc_ref, wpp_ref, wcp_ref, bp_ref, bpp_ref,
               bc_ref, bcp_ref, pl_out, cl_out, pp_out, cp_out):
    xb = x_ref[...]
    pl0 = jax.lax.dot_general(xb, wp_ref[...], _DNT) + bp_ref[...]
    cl_all = jax.lax.dot_general(xb, wc_ref[...], _DNT)

    pln = _ln(pl0)
    pl_out[...] = pln
    pp_out[...] = jax.lax.dot_general(pln, wpp_ref[...], _DNT) + bpp_ref[...]

    pc = jnp.argmax(pl0, axis=1)[:, None]
    lane8 = jax.lax.broadcasted_iota(jnp.int32, (1, NP), 1)
    onehot8 = (pc == lane8).astype(jnp.float32)
    lane128 = jax.lax.broadcasted_iota(jnp.int32, (1, NP * NC), 1)
    mask128 = pc == (lane128 // NC)

    frow = jax.lax.broadcasted_iota(jnp.int32, (NC, NP * NC), 0)
    fcol = jax.lax.broadcasted_iota(jnp.int32, (NC, NP * NC), 1)
    fold = (fcol % NC == frow).astype(jnp.float32)

    clm = jnp.where(mask128, cl_all, 0.0)
    cl16 = jax.lax.dot_general(clm, fold, _DNT)
    cl16 = cl16 + jax.lax.dot(onehot8, bc_ref[...])

    cln = _ln(cl16)
    cl_out[...] = cln

    rep = jax.lax.dot_general(cln, fold.T, _DNT)
    scat = jnp.where(mask128, rep, 0.0)
    cp = jax.lax.dot(scat, wcp_ref[...])
    cp_out[...] = cp + jax.lax.dot(onehot8, bcp_ref[...])


@jax.jit
def kernel(x, Wp, bp, Wpp, bpp, Wc, bc, Wcp, bcp):
    wc_flat = Wc.reshape(NP * NC, D)
    wcp_flat = jnp.transpose(Wcp, (0, 2, 1)).reshape(NP * NC, PE)

    grid = (B // BB,)
    out = pl.pallas_call(
        _hc_kernel,
        grid=grid,
        compiler_params=pltpu.CompilerParams(
            dimension_semantics=("parallel",)),
        in_specs=[
            pl.BlockSpec((BB, D), lambda i: (i, 0)),
            pl.BlockSpec((NP, D), lambda i: (0, 0)),
            pl.BlockSpec((NP * NC, D), lambda i: (0, 0)),
            pl.BlockSpec((PE, NP), lambda i: (0, 0)),
            pl.BlockSpec((NP * NC, PE), lambda i: (0, 0)),
            pl.BlockSpec((1, NP), lambda i: (0, 0)),
            pl.BlockSpec((1, PE), lambda i: (0, 0)),
            pl.BlockSpec((NP, NC), lambda i: (0, 0)),
            pl.BlockSpec((NP, PE), lambda i: (0, 0)),
        ],
        out_specs=[
            pl.BlockSpec((BB, NP), lambda i: (i, 0)),
            pl.BlockSpec((BB, NC), lambda i: (i, 0)),
            pl.BlockSpec((BB, PE), lambda i: (i, 0)),
            pl.BlockSpec((BB, PE), lambda i: (i, 0)),
        ],
        out_shape=[
            jax.ShapeDtypeStruct((B, NP), jnp.float32),
            jax.ShapeDtypeStruct((B, NC), jnp.float32),
            jax.ShapeDtypeStruct((B, PE), jnp.float32),
            jax.ShapeDtypeStruct((B, PE), jnp.float32),
        ],
    )(x, Wp, wc_flat, Wpp, wcp_flat, bp[None, :], bpp[None, :], bc, bcp)
    return (out[0], out[1], out[2], out[3])
